# baseline (speedup 1.0000x reference)
import functools
import jax
import jax.numpy as jnp
from jax import lax
from jax.experimental import pallas as pl
from jax.experimental.pallas import tpu as pltpu
from jax.experimental.pallas import tpu_sc as plsc

_K = 8192
_C = 64
_BETA = 0.25
_B, _H, _W = 8, 32, 32
_HW = _H * _W
_KC = 512
_NCH = _K // _KC


def _argmax_body(z_ref, e_ref, ids_ref, rmax_ref, nsq_ref, s_ref):
    z = z_ref[0]
    nsq = jnp.sum(z * z, axis=0, keepdims=True)
    n = jnp.sqrt(nsq)
    zn = z / jnp.maximum(n, 1e-12)
    nsq_ref[0] = jnp.sum(zn * zn, axis=0, keepdims=True)
    znh = zn.astype(jnp.bfloat16)

    def pass1(k, rmax):
        e_blk = e_ref[pl.ds(k * _KC, _KC), :]
        sim = lax.dot(e_blk, znh,
                      preferred_element_type=jnp.float32)
        s_ref[k] = sim
        return jnp.maximum(rmax, jnp.max(sim, axis=0, keepdims=True))

    rmax = lax.fori_loop(
        0, _NCH, pass1, jnp.full((1, _HW), -jnp.inf, dtype=jnp.float32),
        unroll=8)
    rmax_ref[0] = rmax

    def find(k, ridx):
        iota = lax.broadcasted_iota(jnp.int32, (_KC, _HW), 0) + k * _KC
        bidx = jnp.min(jnp.where(s_ref[k] == rmax, iota, _K),
                       axis=0, keepdims=True)
        return jnp.minimum(ridx, bidx)

    ids_ref[0] = lax.fori_loop(0, _NCH, find,
                               jnp.full((1, _HW), _K, dtype=jnp.int32),
                               unroll=8)


_NC = 2
_NS = 16
_NW = _NC * _NS
_BPW = (_B * _HW) // _NW


def _make_gather():
    mesh = plsc.VectorSubcoreMesh(core_axis_name="c", subcore_axis_name="s")

    @functools.partial(
        pl.kernel, mesh=mesh,
        out_type=jax.ShapeDtypeStruct((_B * _HW, 128), jnp.float32),
        scratch_types=[
            pltpu.VMEM((_BPW,), jnp.int32),
            pltpu.VMEM((_BPW, 128), jnp.float32),
            pltpu.SemaphoreType.DMA,
        ],
    )
    def gather(table_hbm, idx_hbm, out_hbm, idx_v, rows_v, sem):
        wid = lax.axis_index("s") * _NC + lax.axis_index("c")
        base = wid * _BPW
        pltpu.sync_copy(idx_hbm.at[pl.ds(base, _BPW)], idx_v)
        pltpu.async_copy(table_hbm.at[idx_v], rows_v, sem).wait()
        pltpu.sync_copy(rows_v, out_hbm.at[pl.ds(base, _BPW)])

    return gather


def _finish_body(zqt_ref, rmax_ref, nsq_ref, zq_ref, loss_ref):
    b = pl.program_id(0)
    zq = zqt_ref[0][:, :_C].T
    zq_ref[0] = zq
    batch_term = (jnp.sum(zq * zq) - 2.0 * jnp.sum(rmax_ref[0])
                  + jnp.sum(nsq_ref[0]))

    @pl.when(b == 0)
    def _():
        loss_ref[...] = jnp.zeros((1, 1), jnp.float32)

    loss_ref[...] += jnp.full((1, 1), (_BETA / (_B * _HW * _C)),
                              jnp.float32) * batch_term


def kernel(z, embedding):
    zf = z.reshape(_B, _C, _HW)
    ids3, rmax3, nsq3 = pl.pallas_call(
        _argmax_body,
        grid=(_B,),
        in_specs=[
            pl.BlockSpec((1, _C, _HW), lambda b: (b, 0, 0)),
            pl.BlockSpec((_K, _C), lambda b: (0, 0)),
        ],
        out_specs=[
            pl.BlockSpec((1, 1, _HW), lambda b: (b, 0, 0)),
            pl.BlockSpec((1, 1, _HW), lambda b: (b, 0, 0)),
            pl.BlockSpec((1, 1, _HW), lambda b: (b, 0, 0)),
        ],
        out_shape=[
            jax.ShapeDtypeStruct((_B, 1, _HW), jnp.int32),
            jax.ShapeDtypeStruct((_B, 1, _HW), jnp.float32),
            jax.ShapeDtypeStruct((_B, 1, _HW), jnp.float32),
        ],
        scratch_shapes=[pltpu.VMEM((_NCH, _KC, _HW), jnp.float32)],
    )(zf, embedding.astype(jnp.bfloat16))

    ids_flat = ids3.reshape(_B * _HW)
    emb_pad = jnp.pad(embedding, ((0, 0), (0, 128 - _C)))
    zq_tok = _make_gather()(emb_pad, ids_flat)

    zq3, loss = pl.pallas_call(
        _finish_body,
        grid=(_B,),
        in_specs=[
            pl.BlockSpec((1, _HW, 128), lambda b: (b, 0, 0)),
            pl.BlockSpec((1, 1, _HW), lambda b: (b, 0, 0)),
            pl.BlockSpec((1, 1, _HW), lambda b: (b, 0, 0)),
        ],
        out_specs=[
            pl.BlockSpec((1, _C, _HW), lambda b: (b, 0, 0)),
            pl.BlockSpec((1, 1), lambda b: (0, 0)),
        ],
        out_shape=[
            jax.ShapeDtypeStruct((_B, _C, _HW), jnp.float32),
            jax.ShapeDtypeStruct((1, 1), jnp.float32),
        ],
    )(zq_tok.reshape(_B, _HW, 128), rmax3, nsq3)

    z_q_out = zq3.reshape(_B, _C, _H, _W)
    token_ids = ids3.reshape(_B, _H, _W)
    return (z_q_out, loss[0, 0], token_ids)


# speedup vs baseline: 1.0825x; 1.0825x over previous
import jax
import jax.numpy as jnp
from jax import lax
from jax.experimental import pallas as pl
from jax.experimental.pallas import tpu as pltpu

_K = 8192
_C = 64
_BETA = 0.25
_B, _H, _W = 8, 32, 32
_HW = _H * _W
_KC = 512
_NCH = _K // _KC
_UNROLL = 8


def _vq_body(z_ref, e_ref, et_ref, ids_ref, zq_ref, loss_ref, acc_ref):
    b = pl.program_id(0)
    z = z_ref[0]
    nsq = jnp.sum(z * z, axis=0, keepdims=True)
    n = jnp.sqrt(nsq)
    zn = z / jnp.maximum(n, 1e-12)
    zn_sq = jnp.sum(zn * zn)
    znh = zn.astype(jnp.bfloat16)

    def pass1(k, rmax):
        e_blk = e_ref[pl.ds(k * _KC, _KC), :]
        sim = lax.dot(e_blk, znh,
                      preferred_element_type=jnp.float32)
        return jnp.maximum(rmax, jnp.max(sim, axis=0, keepdims=True))

    rmax = lax.fori_loop(
        0, _NCH, pass1, jnp.full((1, _HW), -jnp.inf, dtype=jnp.float32),
        unroll=_UNROLL)

    acc_ref[...] = jnp.zeros((_C + 3, _HW), jnp.float32)

    def pass2(k, _):
        e_blk = e_ref[pl.ds(k * _KC, _KC), :]
        sim = lax.dot(e_blk, znh,
                      preferred_element_type=jnp.float32)
        onehot = (sim == rmax).astype(jnp.bfloat16)
        g_blk = et_ref[:, pl.ds(k * _KC, _KC)]
        acc_ref[...] += lax.dot(g_blk, onehot,
                                preferred_element_type=jnp.float32)
        return 0

    lax.fori_loop(0, _NCH, pass2, 0, unroll=_UNROLL)
    acc = acc_ref[...]
    count = acc[_C + 2:_C + 3]
    ids = (acc[_C:_C + 1] * 128.0 + acc[_C + 1:_C + 2]).astype(jnp.int32)
    ids_ref[0] = ids
    zq_ref[0] = acc[:_C]

    has_tie = jnp.any(count != 1.0)

    @pl.when(has_tie)
    def _():
        def find(k, ridx):
            e_blk = e_ref[pl.ds(k * _KC, _KC), :]
            sim = lax.dot(e_blk, znh,
                          preferred_element_type=jnp.float32)
            iota = lax.broadcasted_iota(jnp.int32, (_KC, _HW), 0) + k * _KC
            bidx = jnp.min(jnp.where(sim == rmax, iota, _K),
                           axis=0, keepdims=True)
            return jnp.minimum(ridx, bidx)

        ids_x = lax.fori_loop(0, _NCH, find,
                              jnp.full((1, _HW), _K, dtype=jnp.int32))
        ids_ref[0] = ids_x
        acc_ref[...] = jnp.zeros((_C + 3, _HW), jnp.float32)

        def rebuild(k, _):
            iota = lax.broadcasted_iota(jnp.int32, (_KC, _HW), 0) + k * _KC
            onehot = (iota == ids_x).astype(jnp.bfloat16)
            g_blk = et_ref[:, pl.ds(k * _KC, _KC)]
            acc_ref[...] += lax.dot(g_blk, onehot,
                                    preferred_element_type=jnp.float32)
            return 0

        lax.fori_loop(0, _NCH, rebuild, 0)
        zq_ref[0] = acc_ref[:_C]

    zq = zq_ref[0]
    batch_term = jnp.sum(zq * zq) - 2.0 * jnp.sum(rmax) + zn_sq

    @pl.when(b == 0)
    def _():
        loss_ref[...] = jnp.zeros((1, 1), jnp.float32)

    loss_ref[...] += jnp.full((1, 1), (_BETA / (_B * _HW * _C)),
                              jnp.float32) * batch_term


def kernel(z, embedding):
    zf = z.reshape(_B, _C, _HW)
    kio = jnp.arange(_K, dtype=jnp.float32)
    et_aug = jnp.concatenate(
        [embedding.T,
         jnp.floor(kio / 128.0)[None, :],
         jnp.mod(kio, 128.0)[None, :],
         jnp.ones((1, _K), jnp.float32)], axis=0).astype(jnp.bfloat16)
    ids3, zq3, loss = pl.pallas_call(
        _vq_body,
        grid=(_B,),
        in_specs=[
            pl.BlockSpec((1, _C, _HW), lambda b: (b, 0, 0)),
            pl.BlockSpec((_K, _C), lambda b: (0, 0)),
            pl.BlockSpec((_C + 3, _K), lambda b: (0, 0)),
        ],
        out_specs=[
            pl.BlockSpec((1, 1, _HW), lambda b: (b, 0, 0)),
            pl.BlockSpec((1, _C, _HW), lambda b: (b, 0, 0)),
            pl.BlockSpec((1, 1), lambda b: (0, 0)),
        ],
        out_shape=[
            jax.ShapeDtypeStruct((_B, 1, _HW), jnp.int32),
            jax.ShapeDtypeStruct((_B, _C, _HW), jnp.float32),
            jax.ShapeDtypeStruct((1, 1), jnp.float32),
        ],
        scratch_shapes=[
            pltpu.VMEM((_C + 3, _HW), jnp.float32),
        ],
    )(zf, embedding.astype(jnp.bfloat16), et_aug)
    z_q_out = zq3.reshape(_B, _C, _H, _W)
    token_ids = ids3.reshape(_B, _H, _W)
    return (z_q_out, loss[0, 0], token_ids)


# speedup vs baseline: 1.6105x; 1.4877x over previous
import jax
import jax.numpy as jnp
from jax import lax
from jax.experimental import pallas as pl
from jax.experimental.pallas import tpu as pltpu

_K = 8192
_C = 64
_BETA = 0.25
_B, _H, _W = 8, 32, 32
_HW = _H * _W
_KC = 512
_NCH = _K // _KC
_UNROLL = 16


def _vq_body(z_ref, e_ref, et_ref, ids_ref, zq_ref, loss_ref, s_ref, acc_ref):
    b = pl.program_id(0)
    z = z_ref[0]
    nsq = jnp.sum(z * z, axis=0, keepdims=True)
    n = jnp.sqrt(nsq)
    zn = z / jnp.maximum(n, 1e-12)
    zn_sq = jnp.sum(zn * zn)
    znh = zn.astype(jnp.bfloat16)

    def pass1(k, rmax):
        e_blk = e_ref[pl.ds(k * _KC, _KC), :]
        sim = lax.dot(e_blk, znh,
                      preferred_element_type=jnp.float32)
        s_ref[k] = sim
        return jnp.maximum(rmax, jnp.max(sim, axis=0, keepdims=True))

    rmax = lax.fori_loop(
        0, _NCH, pass1, jnp.full((1, _HW), -jnp.inf, dtype=jnp.float32),
        unroll=_UNROLL)

    acc_ref[...] = jnp.zeros((_C + 3, _HW), jnp.float32)

    def pass2(k, _):
        onehot = (s_ref[k] == rmax).astype(jnp.bfloat16)
        g_blk = et_ref[:, pl.ds(k * _KC, _KC)]
        acc_ref[...] += lax.dot(g_blk, onehot,
                                preferred_element_type=jnp.float32)
        return 0

    lax.fori_loop(0, _NCH, pass2, 0, unroll=_UNROLL)
    acc = acc_ref[...]
    count = acc[_C + 2:_C + 3]
    ids = (acc[_C:_C + 1] * 128.0 + acc[_C + 1:_C + 2]).astype(jnp.int32)
    ids_ref[0] = ids
    zq_ref[0] = acc[:_C]

    has_tie = jnp.any(count != 1.0)

    @pl.when(has_tie)
    def _():
        def find(k, ridx):
            iota = lax.broadcasted_iota(jnp.int32, (_KC, _HW), 0) + k * _KC
            bidx = jnp.min(jnp.where(s_ref[k] == rmax, iota, _K),
                           axis=0, keepdims=True)
            return jnp.minimum(ridx, bidx)

        ids_x = lax.fori_loop(0, _NCH, find,
                              jnp.full((1, _HW), _K, dtype=jnp.int32))
        ids_ref[0] = ids_x
        acc_ref[...] = jnp.zeros((_C + 3, _HW), jnp.float32)

        def rebuild(k, _):
            iota = lax.broadcasted_iota(jnp.int32, (_KC, _HW), 0) + k * _KC
            onehot = (iota == ids_x).astype(jnp.bfloat16)
            g_blk = et_ref[:, pl.ds(k * _KC, _KC)]
            acc_ref[...] += lax.dot(g_blk, onehot,
                                    preferred_element_type=jnp.float32)
            return 0

        lax.fori_loop(0, _NCH, rebuild, 0)
        zq_ref[0] = acc_ref[:_C]

    zq = zq_ref[0]
    batch_term = jnp.sum(zq * zq) - 2.0 * jnp.sum(rmax) + zn_sq

    @pl.when(b == 0)
    def _():
        loss_ref[...] = jnp.zeros((1, 1), jnp.float32)

    loss_ref[...] += jnp.full((1, 1), (_BETA / (_B * _HW * _C)),
                              jnp.float32) * batch_term


def kernel(z, embedding):
    zf = z.reshape(_B, _C, _HW)
    kio = jnp.arange(_K, dtype=jnp.float32)
    et_aug = jnp.concatenate(
        [embedding.T,
         jnp.floor(kio / 128.0)[None, :],
         jnp.mod(kio, 128.0)[None, :],
         jnp.ones((1, _K), jnp.float32)], axis=0).astype(jnp.bfloat16)
    ids3, zq3, loss = pl.pallas_call(
        _vq_body,
        grid=(_B,),
        in_specs=[
            pl.BlockSpec((1, _C, _HW), lambda b: (b, 0, 0)),
            pl.BlockSpec((_K, _C), lambda b: (0, 0)),
            pl.BlockSpec((_C + 3, _K), lambda b: (0, 0)),
        ],
        out_specs=[
            pl.BlockSpec((1, 1, _HW), lambda b: (b, 0, 0)),
            pl.BlockSpec((1, _C, _HW), lambda b: (b, 0, 0)),
            pl.BlockSpec((1, 1), lambda b: (0, 0)),
        ],
        out_shape=[
            jax.ShapeDtypeStruct((_B, 1, _HW), jnp.int32),
            jax.ShapeDtypeStruct((_B, _C, _HW), jnp.float32),
            jax.ShapeDtypeStruct((1, 1), jnp.float32),
        ],
        scratch_shapes=[
            pltpu.VMEM((_NCH, _KC, _HW), jnp.float32),
            pltpu.VMEM((_C + 3, _HW), jnp.float32),
        ],
    )(zf, embedding.astype(jnp.bfloat16), et_aug)
    z_q_out = zq3.reshape(_B, _C, _H, _W)
    token_ids = ids3.reshape(_B, _H, _W)
    return (z_q_out, loss[0, 0], token_ids)


# speedup vs baseline: 1.6110x; 1.0003x over previous
import jax
import jax.numpy as jnp
from jax import lax
from jax.experimental import pallas as pl
from jax.experimental.pallas import tpu as pltpu

_K = 8192
_C = 64
_BETA = 0.25
_B, _H, _W = 8, 32, 32
_HW = _H * _W
_KC = 1024
_NCH = _K // _KC
_UNROLL = 8


def _vq_body(z_ref, e_ref, et_ref, ids_ref, zq_ref, loss_ref, s_ref, acc_ref):
    b = pl.program_id(0)
    z = z_ref[0]
    nsq = jnp.sum(z * z, axis=0, keepdims=True)
    n = jnp.sqrt(nsq)
    zn = z / jnp.maximum(n, 1e-12)
    zn_sq = jnp.sum(zn * zn)
    znh = zn.astype(jnp.bfloat16)

    def pass1(k, rmax):
        e_blk = e_ref[pl.ds(k * _KC, _KC), :]
        sim = lax.dot(e_blk, znh,
                      preferred_element_type=jnp.float32)
        s_ref[k] = sim
        return jnp.maximum(rmax, jnp.max(sim, axis=0, keepdims=True))

    rmax = lax.fori_loop(
        0, _NCH, pass1, jnp.full((1, _HW), -jnp.inf, dtype=jnp.float32),
        unroll=_UNROLL)

    acc_ref[...] = jnp.zeros((_C + 3, _HW), jnp.float32)

    def pass2(k, _):
        onehot = (s_ref[k] == rmax).astype(jnp.bfloat16)
        g_blk = et_ref[:, pl.ds(k * _KC, _KC)]
        acc_ref[...] += lax.dot(g_blk, onehot,
                                preferred_element_type=jnp.float32)
        return 0

    lax.fori_loop(0, _NCH, pass2, 0, unroll=_UNROLL)
    acc = acc_ref[...]
    count = acc[_C + 2:_C + 3]
    ids = (acc[_C:_C + 1] * 128.0 + acc[_C + 1:_C + 2]).astype(jnp.int32)
    ids_ref[0] = ids
    zq_ref[0] = acc[:_C]

    has_tie = jnp.any(count != 1.0)

    @pl.when(has_tie)
    def _():
        def find(k, ridx):
            iota = lax.broadcasted_iota(jnp.int32, (_KC, _HW), 0) + k * _KC
            bidx = jnp.min(jnp.where(s_ref[k] == rmax, iota, _K),
                           axis=0, keepdims=True)
            return jnp.minimum(ridx, bidx)

        ids_x = lax.fori_loop(0, _NCH, find,
                              jnp.full((1, _HW), _K, dtype=jnp.int32))
        ids_ref[0] = ids_x
        acc_ref[...] = jnp.zeros((_C + 3, _HW), jnp.float32)

        def rebuild(k, _):
            iota = lax.broadcasted_iota(jnp.int32, (_KC, _HW), 0) + k * _KC
            onehot = (iota == ids_x).astype(jnp.bfloat16)
            g_blk = et_ref[:, pl.ds(k * _KC, _KC)]
            acc_ref[...] += lax.dot(g_blk, onehot,
                                    preferred_element_type=jnp.float32)
            return 0

        lax.fori_loop(0, _NCH, rebuild, 0)
        zq_ref[0] = acc_ref[:_C]

    zq = zq_ref[0]
    batch_term = jnp.sum(zq * zq) - 2.0 * jnp.sum(rmax) + zn_sq

    @pl.when(b == 0)
    def _():
        loss_ref[...] = jnp.zeros((1, 1), jnp.float32)

    loss_ref[...] += jnp.full((1, 1), (_BETA / (_B * _HW * _C)),
                              jnp.float32) * batch_term


def kernel(z, embedding):
    zf = z.reshape(_B, _C, _HW)
    kio = jnp.arange(_K, dtype=jnp.float32)
    et_aug = jnp.concatenate(
        [embedding.T,
         jnp.floor(kio / 128.0)[None, :],
         jnp.mod(kio, 128.0)[None, :],
         jnp.ones((1, _K), jnp.float32)], axis=0).astype(jnp.bfloat16)
    ids3, zq3, loss = pl.pallas_call(
        _vq_body,
        grid=(_B,),
        in_specs=[
            pl.BlockSpec((1, _C, _HW), lambda b: (b, 0, 0)),
            pl.BlockSpec((_K, _C), lambda b: (0, 0)),
            pl.BlockSpec((_C + 3, _K), lambda b: (0, 0)),
        ],
        out_specs=[
            pl.BlockSpec((1, 1, _HW), lambda b: (b, 0, 0)),
            pl.BlockSpec((1, _C, _HW), lambda b: (b, 0, 0)),
            pl.BlockSpec((1, 1), lambda b: (0, 0)),
        ],
        out_shape=[
            jax.ShapeDtypeStruct((_B, 1, _HW), jnp.int32),
            jax.ShapeDtypeStruct((_B, _C, _HW), jnp.float32),
            jax.ShapeDtypeStruct((1, 1), jnp.float32),
        ],
        scratch_shapes=[
            pltpu.VMEM((_NCH, _KC, _HW), jnp.float32),
            pltpu.VMEM((_C + 3, _HW), jnp.float32),
        ],
    )(zf, embedding.astype(jnp.bfloat16), et_aug)
    z_q_out = zq3.reshape(_B, _C, _H, _W)
    token_ids = ids3.reshape(_B, _H, _W)
    return (z_q_out, loss[0, 0], token_ids)
